# baseline (speedup 1.0000x reference)
import functools

import jax
import jax.numpy as jnp
from jax import lax
from jax.experimental import pallas as pl
from jax.experimental.pallas import tpu as pltpu
from jax.experimental.pallas import tpu_sc as plsc

_N = 100
_T_EFF = 0.05
_THETA = 0.2

_B = 512
_M = 784
_NC = 2
_NS = 16
_NW = _NC * _NS
_TC_ROWS = 128
_SC_CB = 128
_SC_CHUNKS = (_B - _TC_ROWS) // _SC_CB
_ROWS_PER_W = _SC_CB // _NW
_R = 8
_SLOTS = 8


def _onehot_block(xb):
    mask = xb > _THETA
    ratio = jnp.where(mask, xb / (xb - _THETA), 1.0)
    t = (_T_EFF * jnp.log(ratio) * _N).astype(jnp.int32)
    t = jnp.where(mask, t, -1)
    shape3 = t.shape + (_N,)
    t3 = jax.lax.broadcast_in_dim(t, shape3, (0, 1))
    iota = jax.lax.broadcasted_iota(jnp.int32, shape3, 2)
    return (t3 == iota).astype(jnp.int32)


def _tc_direct(x_ref, o_hbm, ov, sem):
    i = pl.program_id(0)
    nsteps = pl.num_programs(0)
    slot = jax.lax.rem(i, _SLOTS)

    @pl.when(i >= _SLOTS)
    def _wait_prev():
        pltpu.make_async_copy(
            ov.at[slot],
            o_hbm.at[pl.ds((i - _SLOTS) * _R, _R)],
            sem.at[slot],
        ).wait()

    ov[slot] = _onehot_block(x_ref[...])
    pltpu.make_async_copy(
        ov.at[slot],
        o_hbm.at[pl.ds(i * _R, _R)],
        sem.at[slot],
    ).start()

    @pl.when(i == nsteps - 1)
    def _drain():
        for s in range(_SLOTS):
            pltpu.make_async_copy(
                ov.at[s],
                o_hbm.at[pl.ds(0, _R)],
                sem.at[s],
            ).wait()


def _latency_kernel(x_ref, t_ref):
    xb = x_ref[...]
    mask = xb > _THETA
    ratio = jnp.where(mask, xb / (xb - _THETA), 1.0)
    t = (_T_EFF * jnp.log(ratio) * _N).astype(jnp.int32)
    t_ref[...] = jnp.where(mask, t, -1)


def _sc_onehot_body(t_hbm, o_hbm, t_all, buf, sem):
    w = lax.axis_index("s") * _NC + lax.axis_index("c")
    row0 = w * _ROWS_PER_W
    pltpu.sync_copy(t_hbm.at[pl.ds(row0 * _M, _ROWS_PER_W * _M)], t_all)

    lanes = lax.iota(jnp.int32, 16)
    ones = jnp.ones((16,), jnp.int32)
    zeros = jnp.zeros((16,), jnp.int32)

    def zero_row(r, carry):
        rv = jnp.full((16,), r, jnp.int32)
        for j in range(7):
            col = lanes + (j * 16)
            plsc.store_scatter(buf, [rv, col], zeros, mask=col < _N)
        return carry

    lax.fori_loop(0, _M, zero_row, 0)

    def chunk(c, carry):
        tbase = c * _M
        for j in range(_M // 16):
            tv = t_all[pl.ds(tbase + j * 16, 16)]
            valid = (tv >= 0) & (tv < _N)
            row = lanes + (j * 16)
            plsc.store_scatter(buf, [row, tv], ones, mask=valid)
        pltpu.async_copy(buf, o_hbm.at[row0 + c], sem).wait()
        for j in range(_M // 16):
            tv = t_all[pl.ds(tbase + j * 16, 16)]
            valid = (tv >= 0) & (tv < _N)
            row = lanes + (j * 16)
            plsc.store_scatter(buf, [row, tv], zeros, mask=valid)
        return carry

    lax.fori_loop(0, _ROWS_PER_W, chunk, 0)


def kernel(x):
    B, M = x.shape

    out = pl.pallas_call(
        _tc_direct,
        grid=(_TC_ROWS // _R,),
        in_specs=[pl.BlockSpec((_R, M), lambda i: (i, 0))],
        out_specs=pl.BlockSpec(memory_space=pltpu.MemorySpace.HBM),
        out_shape=jax.ShapeDtypeStruct((B, M, _N), jnp.int32),
        scratch_shapes=[
            pltpu.VMEM((_SLOTS, _R, M, _N), jnp.int32),
            pltpu.SemaphoreType.DMA((_SLOTS,)),
        ],
        compiler_params=pltpu.CompilerParams(
            dimension_semantics=("arbitrary",),
        ),
    )(x)

    xs = lax.slice(x, (_TC_ROWS, 0), (B, M))
    xr = jnp.reshape(xs, ((B - _TC_ROWS) * M // 128, 128))
    t = pl.pallas_call(
        _latency_kernel,
        out_shape=jax.ShapeDtypeStruct(xr.shape, jnp.int32),
    )(xr)
    t = jnp.reshape(t, ((B - _TC_ROWS) * M,))

    sc_onehot = functools.partial(
        pl.kernel,
        out_type=jax.ShapeDtypeStruct((_SC_CB, M, _N), jnp.int32),
        mesh=plsc.VectorSubcoreMesh(
            core_axis_name="c", subcore_axis_name="s",
            num_cores=_NC, num_subcores=_NS,
        ),
        scratch_types=[
            pltpu.VMEM((_ROWS_PER_W * M,), jnp.int32),
            pltpu.VMEM((M, _N), jnp.int32),
            pltpu.SemaphoreType.DMA,
        ],
        compiler_params=pltpu.CompilerParams(needs_layout_passes=False),
    )(_sc_onehot_body)

    for k in range(_SC_CHUNKS):
        tk = lax.slice(t, (k * _SC_CB * M,), ((k + 1) * _SC_CB * M,))
        part = sc_onehot(tk)
        out = lax.dynamic_update_slice(out, part, (_TC_ROWS + k * _SC_CB, 0, 0))
    return out


# speedup vs baseline: 1.1823x; 1.1823x over previous
import functools

import jax
import jax.numpy as jnp
from jax import lax
from jax.experimental import pallas as pl
from jax.experimental.pallas import tpu as pltpu
from jax.experimental.pallas import tpu_sc as plsc

_N = 100
_T_EFF = 0.05
_THETA = 0.2

_B = 512
_M = 784
_NC = 2
_NS = 16
_NW = _NC * _NS
_ROWS_PER_W = _B // _NW


def _latency_kernel(x_ref, t_ref):
    xb = x_ref[...]
    mask = xb > _THETA
    ratio = jnp.where(mask, xb / (xb - _THETA), 1.0)
    t = (_T_EFF * jnp.log(ratio) * _N).astype(jnp.int32)
    t_ref[...] = jnp.where(mask, t, -1)


def _sc_onehot_body(t_hbm, o_hbm, t_all, buf, sem):
    w = lax.axis_index("s") * _NC + lax.axis_index("c")
    row0 = w * _ROWS_PER_W
    pltpu.sync_copy(t_hbm.at[pl.ds(row0 * _M, _ROWS_PER_W * _M)], t_all)

    lanes = lax.iota(jnp.int32, 16)
    ones = jnp.ones((16,), jnp.int32)
    zeros = jnp.zeros((16,), jnp.int32)

    def zero_row(r, carry):
        rv = jnp.full((16,), r, jnp.int32)
        for j in range(7):
            col = lanes + (j * 16)
            plsc.store_scatter(buf, [rv, col], zeros, mask=col < _N)
        return carry

    lax.fori_loop(0, _M, zero_row, 0)

    def chunk(c, carry):
        tbase = c * _M
        for j in range(_M // 16):
            tv = t_all[pl.ds(tbase + j * 16, 16)]
            valid = (tv >= 0) & (tv < _N)
            row = lanes + (j * 16)
            plsc.store_scatter(buf, [row, tv], ones, mask=valid)
        pltpu.async_copy(buf, o_hbm.at[row0 + c], sem).wait()
        for j in range(_M // 16):
            tv = t_all[pl.ds(tbase + j * 16, 16)]
            valid = (tv >= 0) & (tv < _N)
            row = lanes + (j * 16)
            plsc.store_scatter(buf, [row, tv], zeros, mask=valid)
        return carry

    lax.fori_loop(0, _ROWS_PER_W, chunk, 0)


def kernel(x):
    B, M = x.shape
    xr = jnp.reshape(x, (B * M // 128, 128))
    t = pl.pallas_call(
        _latency_kernel,
        out_shape=jax.ShapeDtypeStruct(xr.shape, jnp.int32),
    )(xr)
    t = jnp.reshape(t, (B * M,))

    sc_onehot = functools.partial(
        pl.kernel,
        out_type=jax.ShapeDtypeStruct((B, M, _N), jnp.int32),
        mesh=plsc.VectorSubcoreMesh(
            core_axis_name="c", subcore_axis_name="s",
            num_cores=_NC, num_subcores=_NS,
        ),
        scratch_types=[
            pltpu.VMEM((_ROWS_PER_W * M,), jnp.int32),
            pltpu.VMEM((M, _N), jnp.int32),
            pltpu.SemaphoreType.DMA,
        ],
        compiler_params=pltpu.CompilerParams(needs_layout_passes=False),
    )(_sc_onehot_body)
    return sc_onehot(t) | jnp.int32(0)
